# baseline (speedup 1.0000x reference)
import functools

import jax
import jax.numpy as jnp
from jax import lax
from jax.experimental import pallas as pl
from jax.experimental.pallas import tpu as pltpu
from jax.experimental.pallas import tpu_sc as plsc

VOCAB = 50257
EMBED = 768
B_TOTAL = 4 * 2048
NUM_WORKERS = 32
B_PER_W = B_TOTAL // NUM_WORKERS
CHUNK = 32
N_CHUNKS = B_PER_W // CHUNK
NBUF = 4
PRIME = NBUF - 1
WORKERS_PER_ROW = 2048 // B_PER_W


@functools.partial(
    pl.kernel,
    mesh=plsc.VectorSubcoreMesh(core_axis_name="c", subcore_axis_name="s"),
    out_type=jax.ShapeDtypeStruct((B_TOTAL, EMBED), jnp.float32),
    scratch_types=[
        pltpu.VMEM((B_PER_W,), jnp.int32),
        pltpu.VMEM((NBUF, CHUNK, EMBED), jnp.float32),
        pltpu.SemaphoreType.DMA((NBUF,)),
        pltpu.SemaphoreType.DMA((NBUF,)),
    ],
)
def _embed_lookup(table_hbm, idx_hbm, out_hbm, idx_v, rows_v, gsem, wsem):
    c = lax.axis_index("c")
    s = lax.axis_index("s")
    wid = s * 2 + c
    base = wid * B_PER_W

    def gather_start(j, b):
        return pltpu.async_copy(
            table_hbm.at[idx_v.at[pl.ds(j * CHUNK, CHUNK)]],
            rows_v.at[b], gsem.at[b])

    def write_copy(j, b):
        return pltpu.make_async_copy(
            rows_v.at[b], out_hbm.at[pl.ds(base + j * CHUNK, CHUNK)],
            wsem.at[b])

    pltpu.sync_copy(
        idx_hbm.at[wid // WORKERS_PER_ROW,
                   pl.ds((wid % WORKERS_PER_ROW) * B_PER_W, B_PER_W)],
        idx_v)
    for j in range(PRIME):
        gather_start(j, j)

    def body(j, carry):
        b = lax.rem(j, NBUF)
        pltpu.make_async_copy(
            table_hbm.at[idx_v.at[pl.ds(j * CHUNK, CHUNK)]],
            rows_v.at[b], gsem.at[b]).wait()
        write_copy(j, b).start()
        nj = j + PRIME

        @pl.when(nj < N_CHUNKS)
        def _():
            @pl.when(j >= 1)
            def _():
                write_copy(j - 1, lax.rem(j - 1, NBUF)).wait()
            gather_start(nj, lax.rem(nj, NBUF))
        return carry

    lax.fori_loop(0, N_CHUNKS, body, 0)
    for j in range(N_CHUNKS - PRIME - 1, N_CHUNKS):
        write_copy(j, j % NBUF).wait()


def kernel(x, table):
    out = _embed_lookup(table, x.astype(jnp.int32))
    return out.reshape(x.shape[0], x.shape[1], EMBED)


# speedup vs baseline: 1.0090x; 1.0090x over previous
import functools

import jax
import jax.numpy as jnp
from jax import lax
from jax.experimental import pallas as pl
from jax.experimental.pallas import tpu as pltpu
from jax.experimental.pallas import tpu_sc as plsc

VOCAB = 50257
EMBED = 768
B_TOTAL = 4 * 2048
NUM_WORKERS = 32
B_PER_W = B_TOTAL // NUM_WORKERS
CHUNK = 32
N_CHUNKS = B_PER_W // CHUNK
NBUF = 4
PRIME = NBUF - 1
WORKERS_PER_ROW = 2048 // B_PER_W


@functools.partial(
    pl.kernel,
    mesh=plsc.VectorSubcoreMesh(core_axis_name="c", subcore_axis_name="s"),
    out_type=jax.ShapeDtypeStruct((B_TOTAL, EMBED), jnp.float32),
    scratch_types=[
        pltpu.VMEM((B_PER_W,), jnp.int32),
        pltpu.VMEM((NBUF, CHUNK, EMBED), jnp.float32),
        pltpu.SemaphoreType.DMA((NBUF,)),
        pltpu.SemaphoreType.DMA((NBUF,)),
        pltpu.SemaphoreType.DMA,
    ],
)
def _embed_lookup(table_hbm, idx_hbm, out_hbm, idx_v, rows_v, gsem, wsem, isem):
    c = lax.axis_index("c")
    s = lax.axis_index("s")
    wid = s * 2 + c
    base = wid * B_PER_W

    def gather_start(j, b):
        return pltpu.async_copy(
            table_hbm.at[idx_v.at[pl.ds(j * CHUNK, CHUNK)]],
            rows_v.at[b], gsem.at[b])

    def write_copy(j, b):
        return pltpu.make_async_copy(
            rows_v.at[b], out_hbm.at[pl.ds(base + j * CHUNK, CHUNK)],
            wsem.at[b])

    head = PRIME * CHUNK
    pltpu.sync_copy(idx_hbm.at[pl.ds(base, head)], idx_v.at[pl.ds(0, head)])
    idx_rest = pltpu.async_copy(
        idx_hbm.at[pl.ds(base + head, B_PER_W - head)],
        idx_v.at[pl.ds(head, B_PER_W - head)], isem)
    for j in range(PRIME):
        gather_start(j, j)
    idx_rest.wait()

    def body(j, carry):
        b = lax.rem(j, NBUF)
        pltpu.make_async_copy(
            table_hbm.at[idx_v.at[pl.ds(j * CHUNK, CHUNK)]],
            rows_v.at[b], gsem.at[b]).wait()
        write_copy(j, b).start()
        nj = j + PRIME

        @pl.when(nj < N_CHUNKS)
        def _():
            @pl.when(j >= 1)
            def _():
                write_copy(j - 1, lax.rem(j - 1, NBUF)).wait()
            gather_start(nj, lax.rem(nj, NBUF))
        return carry

    lax.fori_loop(0, N_CHUNKS, body, 0)
    for j in range(N_CHUNKS - PRIME - 1, N_CHUNKS):
        write_copy(j, j % NBUF).wait()


def kernel(x, table):
    out = _embed_lookup(table, x.reshape(B_TOTAL).astype(jnp.int32))
    return out.reshape(x.shape[0], x.shape[1], EMBED)


# speedup vs baseline: 1.0102x; 1.0013x over previous
import functools

import jax
import jax.numpy as jnp
from jax import lax
from jax.experimental import pallas as pl
from jax.experimental.pallas import tpu as pltpu
from jax.experimental.pallas import tpu_sc as plsc

VOCAB = 50257
EMBED = 768
B_TOTAL = 4 * 2048
NUM_WORKERS = 32
B_PER_W = B_TOTAL // NUM_WORKERS
CHUNK = 32
N_CHUNKS = B_PER_W // CHUNK
NBUF = 4
PRIME = NBUF - 1


@functools.partial(
    pl.kernel,
    mesh=plsc.VectorSubcoreMesh(core_axis_name="c", subcore_axis_name="s"),
    out_type=jax.ShapeDtypeStruct((B_TOTAL, EMBED), jnp.float32),
    scratch_types=[
        pltpu.VMEM((B_PER_W,), jnp.int32),
        pltpu.VMEM((NBUF, CHUNK, EMBED), jnp.float32),
        pltpu.SemaphoreType.DMA((NBUF,)),
        pltpu.SemaphoreType.DMA((NBUF,)),
        pltpu.SemaphoreType.DMA,
    ],
)
def _embed_lookup(table_hbm, idx_hbm, out_hbm, idx_v, rows_v, gsem, wsem, isem):
    c = lax.axis_index("c")
    s = lax.axis_index("s")
    wid = s * 2 + c
    base = wid * B_PER_W

    def gather_start(j, b):
        return pltpu.async_copy(
            table_hbm.at[idx_v.at[pl.ds(j * CHUNK, CHUNK)]],
            rows_v.at[b], gsem.at[b])

    def write_copy(j, b):
        return pltpu.make_async_copy(
            rows_v.at[b], out_hbm.at[pl.ds(base + j * CHUNK, CHUNK)],
            wsem.at[b])

    head = PRIME * CHUNK
    pltpu.sync_copy(idx_hbm.at[pl.ds(base, head)], idx_v.at[pl.ds(0, head)])
    idx_rest = pltpu.async_copy(
        idx_hbm.at[pl.ds(base + head, B_PER_W - head)],
        idx_v.at[pl.ds(head, B_PER_W - head)], isem)
    for j in range(PRIME):
        gather_start(j, j)
    idx_rest.wait()

    def body(j, carry):
        b = lax.rem(j, NBUF)
        pltpu.make_async_copy(
            table_hbm.at[idx_v.at[pl.ds(j * CHUNK, CHUNK)]],
            rows_v.at[b], gsem.at[b]).wait()
        write_copy(j, b).start()
        nj = j + PRIME

        @pl.when(nj < N_CHUNKS)
        def _():
            @pl.when(j >= 1)
            def _():
                write_copy(j - 1, lax.rem(j - 1, NBUF)).wait()
            gather_start(nj, lax.rem(nj, NBUF))
        return carry

    lax.fori_loop(0, N_CHUNKS, body, 0)
    for j in range(N_CHUNKS - PRIME - 1, N_CHUNKS):
        write_copy(j, j % NBUF).wait()


def kernel(x, table):
    out = _embed_lookup(table, x.reshape(B_TOTAL).astype(jnp.int32))
    return out.reshape(x.shape[0], x.shape[1], EMBED)
